# Initial kernel scaffold; baseline (speedup 1.0000x reference)
import jax
import jax.numpy as jnp
from jax.experimental import pallas as pl


def kernel(ray_indices, points, lidar_to_worlds):
    raise NotImplementedError("write your pallas kernel here")


# speedup vs baseline: 8.4695x; 8.4695x over previous
import functools

import jax
import jax.numpy as jnp
from jax import lax
from jax.experimental import pallas as pl
from jax.experimental.pallas import tpu as pltpu, tpu_sc as plsc

NUM_RAYS = 262144
L = 16
NW = 32
RW = NUM_RAYS // NW


def _rsqrt(s):
    i = plsc.bitcast(s, jnp.int32)
    i = jnp.int32(0x5F3759DF) - lax.shift_right_logical(i, 1)
    y = plsc.bitcast(i, jnp.float32)
    half_s = 0.5 * s
    for _ in range(3):
        y = y * (1.5 - half_s * y * y)
    return y


def _body(ri_hbm, pts_hbm, tbl_hbm, out_hbm, idx_v, pts_v, out_v, tbl_v, sem):
    wid = lax.axis_index("s") * 2 + lax.axis_index("c")
    base = wid * RW
    pltpu.sync_copy(tbl_hbm, tbl_v)
    pltpu.sync_copy(ri_hbm.at[pl.ds(base * 2, RW * 2)], idx_v)
    pltpu.sync_copy(pts_hbm.at[pl.ds(base * 3, RW * 3)], pts_v)

    lane = lax.iota(jnp.int32, L)

    def step(v, carry):
        r = v * L + lane
        c = plsc.load_gather(idx_v, [r * 2])
        r3 = r * 3
        px = plsc.load_gather(pts_v, [r3])
        py = plsc.load_gather(pts_v, [r3 + 1])
        pz = plsc.load_gather(pts_v, [r3 + 2])
        cb = c * 12
        g = lambda k: plsc.load_gather(tbl_v, [cb + k])
        dx = g(0) * px + g(1) * py + g(2) * pz
        dy = g(4) * px + g(5) * py + g(6) * pz
        dz = g(8) * px + g(9) * py + g(10) * pz
        s = jnp.maximum(dx * dx + dy * dy + dz * dz, 1e-16)
        inv = _rsqrt(s)
        r6 = r * 6
        plsc.store_scatter(out_v, [r6], g(3))
        plsc.store_scatter(out_v, [r6 + 1], g(7))
        plsc.store_scatter(out_v, [r6 + 2], g(11))
        plsc.store_scatter(out_v, [r6 + 3], dx * inv)
        plsc.store_scatter(out_v, [r6 + 4], dy * inv)
        plsc.store_scatter(out_v, [r6 + 5], dz * inv)
        return carry

    lax.fori_loop(0, RW // L, step, 0)
    pltpu.sync_copy(out_v, out_hbm.at[pl.ds(base * 6, RW * 6)])


@jax.jit
def _run(ri_flat, pts_flat, tbl_flat):
    mesh = plsc.VectorSubcoreMesh(core_axis_name="c", subcore_axis_name="s")
    return pl.kernel(
        _body,
        out_type=jax.ShapeDtypeStruct((NUM_RAYS * 6,), jnp.float32),
        mesh=mesh,
        compiler_params=pltpu.CompilerParams(needs_layout_passes=False),
        scratch_types=[
            pltpu.VMEM((RW * 2,), jnp.int32),
            pltpu.VMEM((RW * 3,), jnp.float32),
            pltpu.VMEM((RW * 6,), jnp.float32),
            pltpu.VMEM((96,), jnp.float32),
            pltpu.SemaphoreType.DMA,
        ],
    )(ri_flat, pts_flat, tbl_flat)


def kernel(ray_indices, points, lidar_to_worlds):
    ri = ray_indices.astype(jnp.int32).reshape(-1)
    out = _run(ri, points.reshape(-1), lidar_to_worlds.reshape(-1))
    return out.reshape(NUM_RAYS, 6)
